# baseline (speedup 1.0000x reference)
import functools

import jax
import jax.numpy as jnp
from jax import lax
from jax.experimental import pallas as pl
from jax.experimental.pallas import tpu as pltpu
from jax.experimental.pallas import tpu_sc as plsc

NC, NS, L = 2, 16, 16
NW = NC * NS
G = 16
H0 = 8


def _sc_body(B, H, S, KMAX, mask_hbm, probs_hbm, out_hbm,
             mask_v, kept_v, rows_v, wts_v, idx_v, buf_v, buf2_v, acc_v,
             sem, sem2):
    cid = lax.axis_index("c")
    sid = lax.axis_index("s")
    gwid = cid * NS + sid

    pltpu.sync_copy(mask_hbm, mask_v)

    zero16 = jnp.zeros((L,), jnp.float32)
    for b in range(B):
        for c in range(S // L):
            acc_v[b, pl.ds(c * L, L)] = zero16

    lane = lax.broadcasted_iota(jnp.int32, (L,), 0)

    for b in range(B):
        def comp_body(c, off):
            m = mask_v[pl.ds(b * S + c * L, L)] >= 0.0
            mi = m.astype(jnp.int32)
            pos = jnp.broadcast_to(off, (L,)) + plsc.cumsum(mi) - 1
            ids = lane + c * L
            plsc.store_scatter(kept_v, (pos,), ids, mask=m)
            return off + jnp.sum(mi)

        cnt = lax.fori_loop(0, S // L, comp_body, jnp.int32(0))

        gwid_vec = jnp.broadcast_to(gwid, (L,))
        cnt_vec = jnp.broadcast_to(cnt, (L,))
        for k in range(KMAX // L):
            p_vec = gwid_vec + (NW * (k * L) + NW * lane)
            g = plsc.load_gather(kept_v, (p_vec,))
            rows_v[b, pl.ds(k * L, L)] = jnp.clip(g, 0, S - 1)
            wts_v[b, pl.ds(k * L, L)] = (p_vec < cnt_vec).astype(jnp.float32)

        my_cnt = jnp.maximum(0, (cnt - gwid + NW - 1) // NW)
        n_chunks = (my_cnt + G - 1) // G

        for hi in range(H - H0):
            h = H0 + hi
            for k in range(KMAX // L):
                idx_v[pl.ds(hi * KMAX + k * L, L)] = (
                    rows_v[b, pl.ds(k * L, L)] + (b * H + h) * S)

        total = (H - H0) * n_chunks

        def off_of(t):
            return (t // n_chunks) * KMAX + (t % n_chunks) * G

        def start_chunk(t_off, buf, sem):
            pltpu.async_copy(
                probs_hbm.at[idx_v.at[pl.ds(t_off, G)]], buf, sem)

        def wait_chunk(t_off, buf, sem):
            pltpu.make_async_copy(
                probs_hbm.at[idx_v.at[pl.ds(t_off, G)]], buf, sem).wait()

        def accum(buf, t, b=b):
            c = t % n_chunks
            bv = jnp.full((L,), b, jnp.int32)
            ws = []
            for g in range(G):
                kv = jnp.broadcast_to(c * G + g, (L,))
                ws.append(plsc.load_gather(wts_v, (bv, kv)))

            @plsc.parallel_loop(0, S // L, 1, unroll=8)
            def _(i):
                sl = pl.ds(i * L, L)
                t = [buf[g, sl] * ws[g] + buf[g + 1, sl] * ws[g + 1]
                     for g in range(0, G, 2)]
                while len(t) > 1:
                    t = [t[j] + t[j + 1] for j in range(0, len(t) - 1, 2)
                         ] + ([t[-1]] if len(t) % 2 else [])
                plsc.addupdate(acc_v.at[b, sl], t[0])

        @pl.when(total > 0)
        def _():
            start_chunk(0, buf_v, sem)

        def pair_body(p, _):
            t0 = 2 * p
            start_chunk(off_of(t0 + 1), buf2_v, sem2)
            wait_chunk(off_of(t0), buf_v, sem)
            accum(buf_v, t0)

            @pl.when(t0 + 2 < total)
            def _():
                start_chunk(off_of(t0 + 2), buf_v, sem)

            wait_chunk(off_of(t0 + 1), buf2_v, sem2)
            accum(buf2_v, t0 + 1)
            return 0

        lax.fori_loop(0, total // 2, pair_body, 0)

        @pl.when(total % 2 == 1)
        def _():
            t = total - 1
            wait_chunk(off_of(t), buf_v, sem)
            accum(buf_v, t)

    pltpu.sync_copy(acc_v, out_hbm.at[gwid])


def _tc_body(H0_, NB, mask_ref, probs_ref, sums_ref):
    h = pl.program_id(1)
    nb = pl.program_id(2)
    w = (mask_ref[0, :, 0] >= 0).astype(jnp.float32)
    part = jnp.dot(w[None, :], probs_ref[0, 0],
                   preferred_element_type=jnp.float32)[None]
    first = jnp.logical_and(h == 0, nb == 0)

    @pl.when(first)
    def _():
        sums_ref[...] = part

    @pl.when(jnp.logical_not(first))
    def _():
        sums_ref[...] += part


def _finish_body(inv_n, thr_ref, parts_ref, tc_ref, scores_ref, newmask_ref):
    s = (jnp.sum(parts_ref[...], axis=0) + tc_ref[:, 0, :]) * inv_n
    scores_ref[...] = s
    newmask_ref[...] = jnp.where(s < thr_ref[0, 0], -10000.0, 0.0)


def kernel(attention_mask, attention_probs, sentence_lengths, keep_threshold):
    B, H, S, _ = attention_probs.shape
    KMAX = S // NW
    BS = 512
    NB = S // BS
    mask_flat = attention_mask.reshape(B * S)
    mask3 = attention_mask.reshape(B, S, 1)
    probs_flat = attention_probs.reshape(B * H * S, S)
    thr = jnp.maximum(jnp.float32(1e-5), keep_threshold).reshape(1, 1)

    mesh = plsc.VectorSubcoreMesh(core_axis_name="c", subcore_axis_name="s")
    sc = pl.kernel(
        functools.partial(_sc_body, B, H, S, KMAX),
        out_type=jax.ShapeDtypeStruct((NW, B, S), jnp.float32),
        mesh=mesh,
        scratch_types=[
            pltpu.VMEM((B * S,), jnp.float32),
            pltpu.VMEM((S,), jnp.int32),
            pltpu.VMEM((B, KMAX), jnp.int32),
            pltpu.VMEM((B, KMAX), jnp.float32),
            pltpu.VMEM(((H - H0) * KMAX,), jnp.int32),
            pltpu.VMEM((G, S), jnp.float32),
            pltpu.VMEM((G, S), jnp.float32),
            pltpu.VMEM((B, S), jnp.float32),
            pltpu.SemaphoreType.DMA,
            pltpu.SemaphoreType.DMA,
        ],
        compiler_params=pltpu.CompilerParams(needs_layout_passes=False),
    )
    partials = sc(mask_flat, probs_flat)

    tc_sums = pl.pallas_call(
        functools.partial(_tc_body, H0, NB),
        grid=(B, H0, NB),
        in_specs=[
            pl.BlockSpec((1, BS, 1), lambda b, h, nb: (b, nb, 0)),
            pl.BlockSpec((1, 1, BS, S), lambda b, h, nb: (b, h, nb, 0)),
        ],
        out_specs=pl.BlockSpec((1, 1, S), lambda b, h, nb: (b, 0, 0)),
        out_shape=jax.ShapeDtypeStruct((B, 1, S), jnp.float32),
        compiler_params=pltpu.CompilerParams(
            dimension_semantics=("arbitrary", "arbitrary", "arbitrary"),
        ),
    )(mask3, attention_probs)

    scores, newmask = pl.pallas_call(
        functools.partial(_finish_body, 1.0 / (H * S)),
        in_specs=[
            pl.BlockSpec(memory_space=pltpu.SMEM),
            pl.BlockSpec((NW, B, S), lambda: (0, 0, 0)),
            pl.BlockSpec((B, 1, S), lambda: (0, 0, 0)),
        ],
        out_specs=[
            pl.BlockSpec((B, S), lambda: (0, 0)),
            pl.BlockSpec((B, S), lambda: (0, 0)),
        ],
        out_shape=[
            jax.ShapeDtypeStruct((B, S), jnp.float32),
            jax.ShapeDtypeStruct((B, S), jnp.float32),
        ],
    )(thr, partials, tc_sums)

    return (newmask.reshape(B, 1, 1, S), keep_threshold, scores)


# speedup vs baseline: 1.0777x; 1.0777x over previous
import functools

import jax
import jax.numpy as jnp
from jax import lax
from jax.experimental import pallas as pl
from jax.experimental.pallas import tpu as pltpu
from jax.experimental.pallas import tpu_sc as plsc

NC, NS, L = 2, 16, 16
NW = NC * NS
G = 16
H0 = 7


def _sc_body(B, H, S, KMAX, mask_hbm, probs_hbm, out_hbm,
             mask_v, kept_v, rows_v, wts_v, idx_v, buf_v, buf2_v, acc_v,
             sem, sem2):
    cid = lax.axis_index("c")
    sid = lax.axis_index("s")
    gwid = cid * NS + sid

    pltpu.sync_copy(mask_hbm, mask_v)

    zero16 = jnp.zeros((L,), jnp.float32)
    for b in range(B):
        for c in range(S // L):
            acc_v[b, pl.ds(c * L, L)] = zero16

    lane = lax.broadcasted_iota(jnp.int32, (L,), 0)

    for b in range(B):
        def comp_body(c, off):
            m = mask_v[pl.ds(b * S + c * L, L)] >= 0.0
            mi = m.astype(jnp.int32)
            pos = jnp.broadcast_to(off, (L,)) + plsc.cumsum(mi) - 1
            ids = lane + c * L
            plsc.store_scatter(kept_v, (pos,), ids, mask=m)
            return off + jnp.sum(mi)

        cnt = lax.fori_loop(0, S // L, comp_body, jnp.int32(0))

        gwid_vec = jnp.broadcast_to(gwid, (L,))
        cnt_vec = jnp.broadcast_to(cnt, (L,))
        for k in range(KMAX // L):
            p_vec = gwid_vec + (NW * (k * L) + NW * lane)
            g = plsc.load_gather(kept_v, (p_vec,))
            rows_v[b, pl.ds(k * L, L)] = jnp.clip(g, 0, S - 1)
            wts_v[b, pl.ds(k * L, L)] = (p_vec < cnt_vec).astype(jnp.float32)

        my_cnt = jnp.maximum(0, (cnt - gwid + NW - 1) // NW)
        n_chunks = (my_cnt + G - 1) // G

        for hi in range(H - H0):
            h = H0 + hi
            for k in range(KMAX // L):
                idx_v[pl.ds(hi * KMAX + k * L, L)] = (
                    rows_v[b, pl.ds(k * L, L)] + (b * H + h) * S)

        total = (H - H0) * n_chunks

        def off_of(t):
            return (t // n_chunks) * KMAX + (t % n_chunks) * G

        def start_chunk(t_off, buf, sem):
            pltpu.async_copy(
                probs_hbm.at[idx_v.at[pl.ds(t_off, G)]], buf, sem)

        def wait_chunk(t_off, buf, sem):
            pltpu.make_async_copy(
                probs_hbm.at[idx_v.at[pl.ds(t_off, G)]], buf, sem).wait()

        def accum(buf, t, b=b):
            c = t % n_chunks
            bv = jnp.full((L,), b, jnp.int32)
            ws = []
            for g in range(G):
                kv = jnp.broadcast_to(c * G + g, (L,))
                ws.append(plsc.load_gather(wts_v, (bv, kv)))

            @plsc.parallel_loop(0, S // L, 1, unroll=8)
            def _(i):
                sl = pl.ds(i * L, L)
                t = [buf[g, sl] * ws[g] + buf[g + 1, sl] * ws[g + 1]
                     for g in range(0, G, 2)]
                while len(t) > 1:
                    t = [t[j] + t[j + 1] for j in range(0, len(t) - 1, 2)
                         ] + ([t[-1]] if len(t) % 2 else [])
                plsc.addupdate(acc_v.at[b, sl], t[0])

        @pl.when(total > 0)
        def _():
            start_chunk(0, buf_v, sem)

        def pair_body(p, _):
            t0 = 2 * p
            start_chunk(off_of(t0 + 1), buf2_v, sem2)
            wait_chunk(off_of(t0), buf_v, sem)
            accum(buf_v, t0)

            @pl.when(t0 + 2 < total)
            def _():
                start_chunk(off_of(t0 + 2), buf_v, sem)

            wait_chunk(off_of(t0 + 1), buf2_v, sem2)
            accum(buf2_v, t0 + 1)
            return 0

        lax.fori_loop(0, total // 2, pair_body, 0)

        @pl.when(total % 2 == 1)
        def _():
            t = total - 1
            wait_chunk(off_of(t), buf_v, sem)
            accum(buf_v, t)

    pltpu.sync_copy(acc_v, out_hbm.at[gwid])


def _tc_body(H0_, NB, mask_ref, probs_ref, sums_ref):
    h = pl.program_id(1)
    nb = pl.program_id(2)
    w = (mask_ref[0, :, 0] >= 0).astype(jnp.float32)
    part = jnp.dot(w[None, :], probs_ref[0, 0],
                   preferred_element_type=jnp.float32)[None]
    first = jnp.logical_and(h == 0, nb == 0)

    @pl.when(first)
    def _():
        sums_ref[...] = part

    @pl.when(jnp.logical_not(first))
    def _():
        sums_ref[...] += part


def _finish_body(inv_n, thr_ref, parts_ref, tc_ref, scores_ref, newmask_ref):
    s = (jnp.sum(parts_ref[...], axis=0) + tc_ref[:, 0, :]) * inv_n
    scores_ref[...] = s
    newmask_ref[...] = jnp.where(s < thr_ref[0, 0], -10000.0, 0.0)


def kernel(attention_mask, attention_probs, sentence_lengths, keep_threshold):
    B, H, S, _ = attention_probs.shape
    KMAX = S // NW
    BS = 1024
    NB = S // BS
    mask_flat = attention_mask.reshape(B * S)
    mask3 = attention_mask.reshape(B, S, 1)
    probs_flat = attention_probs.reshape(B * H * S, S)
    thr = jnp.maximum(jnp.float32(1e-5), keep_threshold).reshape(1, 1)

    mesh = plsc.VectorSubcoreMesh(core_axis_name="c", subcore_axis_name="s")
    sc = pl.kernel(
        functools.partial(_sc_body, B, H, S, KMAX),
        out_type=jax.ShapeDtypeStruct((NW, B, S), jnp.float32),
        mesh=mesh,
        scratch_types=[
            pltpu.VMEM((B * S,), jnp.float32),
            pltpu.VMEM((S,), jnp.int32),
            pltpu.VMEM((B, KMAX), jnp.int32),
            pltpu.VMEM((B, KMAX), jnp.float32),
            pltpu.VMEM(((H - H0) * KMAX,), jnp.int32),
            pltpu.VMEM((G, S), jnp.float32),
            pltpu.VMEM((G, S), jnp.float32),
            pltpu.VMEM((B, S), jnp.float32),
            pltpu.SemaphoreType.DMA,
            pltpu.SemaphoreType.DMA,
        ],
        compiler_params=pltpu.CompilerParams(needs_layout_passes=False),
    )
    partials = sc(mask_flat, probs_flat)

    tc_sums = pl.pallas_call(
        functools.partial(_tc_body, H0, NB),
        grid=(B, H0, NB),
        in_specs=[
            pl.BlockSpec((1, BS, 1), lambda b, h, nb: (b, nb, 0)),
            pl.BlockSpec((1, 1, BS, S), lambda b, h, nb: (b, h, nb, 0)),
        ],
        out_specs=pl.BlockSpec((1, 1, S), lambda b, h, nb: (b, 0, 0)),
        out_shape=jax.ShapeDtypeStruct((B, 1, S), jnp.float32),
        compiler_params=pltpu.CompilerParams(
            dimension_semantics=("arbitrary", "arbitrary", "arbitrary"),
        ),
    )(mask3, attention_probs)

    scores, newmask = pl.pallas_call(
        functools.partial(_finish_body, 1.0 / (H * S)),
        in_specs=[
            pl.BlockSpec(memory_space=pltpu.SMEM),
            pl.BlockSpec((NW, B, S), lambda: (0, 0, 0)),
            pl.BlockSpec((B, 1, S), lambda: (0, 0, 0)),
        ],
        out_specs=[
            pl.BlockSpec((B, S), lambda: (0, 0)),
            pl.BlockSpec((B, S), lambda: (0, 0)),
        ],
        out_shape=[
            jax.ShapeDtypeStruct((B, S), jnp.float32),
            jax.ShapeDtypeStruct((B, S), jnp.float32),
        ],
    )(thr, partials, tc_sums)

    return (newmask.reshape(B, 1, 1, S), keep_threshold, scores)


# speedup vs baseline: 1.0901x; 1.0116x over previous
import functools

import jax
import jax.numpy as jnp
from jax import lax
from jax.experimental import pallas as pl
from jax.experimental.pallas import tpu as pltpu
from jax.experimental.pallas import tpu_sc as plsc

NC, NS, L = 2, 16, 16
NW = NC * NS
G = 16
H0 = 6


def _sc_body(B, H, S, KMAX, mask_hbm, probs_hbm, out_hbm,
             mask_v, kept_v, rows_v, wts_v, idx_v, buf_v, buf2_v, acc_v,
             sem, sem2):
    cid = lax.axis_index("c")
    sid = lax.axis_index("s")
    gwid = cid * NS + sid

    pltpu.sync_copy(mask_hbm, mask_v)

    zero16 = jnp.zeros((L,), jnp.float32)
    for b in range(B):
        for c in range(S // L):
            acc_v[b, pl.ds(c * L, L)] = zero16

    lane = lax.broadcasted_iota(jnp.int32, (L,), 0)

    for b in range(B):
        def comp_body(c, off):
            m = mask_v[pl.ds(b * S + c * L, L)] >= 0.0
            mi = m.astype(jnp.int32)
            pos = jnp.broadcast_to(off, (L,)) + plsc.cumsum(mi) - 1
            ids = lane + c * L
            plsc.store_scatter(kept_v, (pos,), ids, mask=m)
            return off + jnp.sum(mi)

        cnt = lax.fori_loop(0, S // L, comp_body, jnp.int32(0))

        gwid_vec = jnp.broadcast_to(gwid, (L,))
        cnt_vec = jnp.broadcast_to(cnt, (L,))
        for k in range(KMAX // L):
            p_vec = gwid_vec + (NW * (k * L) + NW * lane)
            g = plsc.load_gather(kept_v, (p_vec,))
            rows_v[b, pl.ds(k * L, L)] = jnp.clip(g, 0, S - 1)
            wts_v[b, pl.ds(k * L, L)] = (p_vec < cnt_vec).astype(jnp.float32)

        my_cnt = jnp.maximum(0, (cnt - gwid + NW - 1) // NW)
        n_chunks = (my_cnt + G - 1) // G

        for hi in range(H - H0):
            h = H0 + hi
            for k in range(KMAX // L):
                idx_v[pl.ds(hi * KMAX + k * L, L)] = (
                    rows_v[b, pl.ds(k * L, L)] + (b * H + h) * S)

        total = (H - H0) * n_chunks

        def off_of(t):
            return (t // n_chunks) * KMAX + (t % n_chunks) * G

        def start_chunk(t_off, buf, sem):
            pltpu.async_copy(
                probs_hbm.at[idx_v.at[pl.ds(t_off, G)]], buf, sem)

        def wait_chunk(t_off, buf, sem):
            pltpu.make_async_copy(
                probs_hbm.at[idx_v.at[pl.ds(t_off, G)]], buf, sem).wait()

        def accum(buf, t, b=b):
            c = t % n_chunks
            bv = jnp.full((L,), b, jnp.int32)
            ws = []
            for g in range(G):
                kv = jnp.broadcast_to(c * G + g, (L,))
                ws.append(plsc.load_gather(wts_v, (bv, kv)))

            @plsc.parallel_loop(0, S // L, 1, unroll=8)
            def _(i):
                sl = pl.ds(i * L, L)
                t = [buf[g, sl] * ws[g] + buf[g + 1, sl] * ws[g + 1]
                     for g in range(0, G, 2)]
                while len(t) > 1:
                    t = [t[j] + t[j + 1] for j in range(0, len(t) - 1, 2)
                         ] + ([t[-1]] if len(t) % 2 else [])
                plsc.addupdate(acc_v.at[b, sl], t[0])

        @pl.when(total > 0)
        def _():
            start_chunk(0, buf_v, sem)

        def pair_body(p, _):
            t0 = 2 * p
            start_chunk(off_of(t0 + 1), buf2_v, sem2)
            wait_chunk(off_of(t0), buf_v, sem)
            accum(buf_v, t0)

            @pl.when(t0 + 2 < total)
            def _():
                start_chunk(off_of(t0 + 2), buf_v, sem)

            wait_chunk(off_of(t0 + 1), buf2_v, sem2)
            accum(buf2_v, t0 + 1)
            return 0

        lax.fori_loop(0, total // 2, pair_body, 0)

        @pl.when(total % 2 == 1)
        def _():
            t = total - 1
            wait_chunk(off_of(t), buf_v, sem)
            accum(buf_v, t)

    pltpu.sync_copy(acc_v, out_hbm.at[gwid])


def _tc_body(H0_, NB, mask_ref, probs_ref, sums_ref):
    h = pl.program_id(1)
    nb = pl.program_id(2)
    w = (mask_ref[0, :, 0] >= 0).astype(jnp.float32)
    part = jnp.dot(w[None, :], probs_ref[0, 0],
                   preferred_element_type=jnp.float32)[None]
    first = jnp.logical_and(h == 0, nb == 0)

    @pl.when(first)
    def _():
        sums_ref[...] = part

    @pl.when(jnp.logical_not(first))
    def _():
        sums_ref[...] += part


def _finish_body(inv_n, thr_ref, parts_ref, tc_ref, scores_ref, newmask_ref):
    s = (jnp.sum(parts_ref[...], axis=0) + tc_ref[:, 0, :]) * inv_n
    scores_ref[...] = s
    newmask_ref[...] = jnp.where(s < thr_ref[0, 0], -10000.0, 0.0)


def kernel(attention_mask, attention_probs, sentence_lengths, keep_threshold):
    B, H, S, _ = attention_probs.shape
    KMAX = S // NW
    BS = 1024
    NB = S // BS
    mask_flat = attention_mask.reshape(B * S)
    mask3 = attention_mask.reshape(B, S, 1)
    probs_flat = attention_probs.reshape(B * H * S, S)
    thr = jnp.maximum(jnp.float32(1e-5), keep_threshold).reshape(1, 1)

    mesh = plsc.VectorSubcoreMesh(core_axis_name="c", subcore_axis_name="s")
    sc = pl.kernel(
        functools.partial(_sc_body, B, H, S, KMAX),
        out_type=jax.ShapeDtypeStruct((NW, B, S), jnp.float32),
        mesh=mesh,
        scratch_types=[
            pltpu.VMEM((B * S,), jnp.float32),
            pltpu.VMEM((S,), jnp.int32),
            pltpu.VMEM((B, KMAX), jnp.int32),
            pltpu.VMEM((B, KMAX), jnp.float32),
            pltpu.VMEM(((H - H0) * KMAX,), jnp.int32),
            pltpu.VMEM((G, S), jnp.float32),
            pltpu.VMEM((G, S), jnp.float32),
            pltpu.VMEM((B, S), jnp.float32),
            pltpu.SemaphoreType.DMA,
            pltpu.SemaphoreType.DMA,
        ],
        compiler_params=pltpu.CompilerParams(needs_layout_passes=False),
    )
    partials = sc(mask_flat, probs_flat)

    tc_sums = pl.pallas_call(
        functools.partial(_tc_body, H0, NB),
        grid=(B, H0, NB),
        in_specs=[
            pl.BlockSpec((1, BS, 1), lambda b, h, nb: (b, nb, 0)),
            pl.BlockSpec((1, 1, BS, S), lambda b, h, nb: (b, h, nb, 0)),
        ],
        out_specs=pl.BlockSpec((1, 1, S), lambda b, h, nb: (b, 0, 0)),
        out_shape=jax.ShapeDtypeStruct((B, 1, S), jnp.float32),
        compiler_params=pltpu.CompilerParams(
            dimension_semantics=("arbitrary", "arbitrary", "arbitrary"),
        ),
    )(mask3, attention_probs)

    scores, newmask = pl.pallas_call(
        functools.partial(_finish_body, 1.0 / (H * S)),
        in_specs=[
            pl.BlockSpec(memory_space=pltpu.SMEM),
            pl.BlockSpec((NW, B, S), lambda: (0, 0, 0)),
            pl.BlockSpec((B, 1, S), lambda: (0, 0, 0)),
        ],
        out_specs=[
            pl.BlockSpec((B, S), lambda: (0, 0)),
            pl.BlockSpec((B, S), lambda: (0, 0)),
        ],
        out_shape=[
            jax.ShapeDtypeStruct((B, S), jnp.float32),
            jax.ShapeDtypeStruct((B, S), jnp.float32),
        ],
    )(thr, partials, tc_sums)

    return (newmask.reshape(B, 1, 1, S), keep_threshold, scores)
